# baseline (speedup 1.0000x reference)
import functools

import jax
import jax.numpy as jnp
from jax import lax
from jax.experimental import pallas as pl
from jax.experimental.pallas import tpu as pltpu
from jax.experimental.pallas import tpu_sc as plsc

B = 4096
N = 50
D = 64
IDX_MINOR = 100


def _entity_kernel(nc, ns, nl):
    nw = nc * ns
    rows_per_w = B // nw
    cb = nl
    n_chunks = rows_per_w // cb
    g_rows = cb * N
    n_sub = g_rows // IDX_MINOR
    nq = (N + nl - 1) // nl
    tail = N - (nq - 1) * nl
    mesh = plsc.VectorSubcoreMesh(core_axis_name="c", subcore_axis_name="s")

    @functools.partial(
        pl.kernel,
        out_type=(
            jax.ShapeDtypeStruct((B * N,), jnp.float32),
            jax.ShapeDtypeStruct((B * N,), jnp.float32),
        ),
        mesh=mesh,
        compiler_params=pltpu.CompilerParams(
            needs_layout_passes=False, use_tc_tiling_on_sc=False),
        scratch_types=[
            pltpu.VMEM((n_sub, IDX_MINOR), jnp.int32),
            pltpu.VMEM((g_rows, D), jnp.float32),
            pltpu.VMEM((cb, D), jnp.float32),
            pltpu.VMEM((g_rows,), jnp.float32),
            pltpu.VMEM((g_rows,), jnp.float32),
            pltpu.SemaphoreType.DMA,
        ],
    )
    def entity_kernel(ids_hbm, ctx_hbm, table_hbm, scores_hbm, post_hbm,
                      idx_v, rows_v, ctx_v, fs_v, fp_v, sem):
        wid = lax.axis_index("s") * nc + lax.axis_index("c")
        lane = lax.iota(jnp.int32, nl)
        neg_inf = jnp.float32(-jnp.inf)

        def chunk_body(g, _):
            c = wid * n_chunks + g
            fbase = c * g_rows

            pltpu.sync_copy(ids_hbm.at[pl.ds(c * n_sub, n_sub)], idx_v)
            copies = [
                pltpu.async_copy(
                    table_hbm.at[idx_v.at[j]],
                    rows_v.at[pl.ds(j * IDX_MINOR, IDX_MINOR)],
                    sem,
                )
                for j in range(n_sub)
            ]
            pltpu.sync_copy(ctx_hbm.at[pl.ds(c * cb, cb)], ctx_v)
            for cp in copies:
                cp.wait()

            def row_body(b, _):
                rbase = b * N
                obase = b * N
                cvec = [ctx_v[b, pl.ds(k * nl, nl)] for k in range(D // nl)]

                svs = []
                for q in range(nq):
                    acc = jnp.zeros((nl,), jnp.float32)
                    nlim = tail if q == nq - 1 else nl
                    for j in range(nlim):
                        r = rbase + q * nl + j
                        p = rows_v[r, pl.ds(0, nl)] * cvec[0]
                        for k in range(1, D // nl):
                            p = p + rows_v[r, pl.ds(k * nl, nl)] * cvec[k]
                        s = jnp.sum(p)
                        acc = jnp.where(lane == j, s, acc)
                    svs.append(acc)

                mvec = jnp.where(lane < tail, svs[nq - 1], neg_inf)
                for q in range(nq - 1):
                    mvec = jnp.maximum(mvec, svs[q])
                m = jnp.max(mvec)
                evs = [jnp.exp(sv - m) for sv in svs]
                evs[nq - 1] = jnp.where(lane < tail, evs[nq - 1], 0.0)
                ssum = evs[0]
                for q in range(1, nq):
                    ssum = ssum + evs[q]
                svec = jnp.zeros((nl,), jnp.float32) + jnp.sum(ssum)
                rinv = jnp.ones((nl,), jnp.float32) / svec

                for q in range(nq - 1):
                    fs_v[pl.ds(obase + q * nl, nl)] = svs[q]
                    fp_v[pl.ds(obase + q * nl, nl)] = evs[q] * rinv
                tmask = lane < tail
                tidx = obase + (nq - 1) * nl + lane
                plsc.store_scatter(fs_v, [tidx], svs[nq - 1], mask=tmask)
                plsc.store_scatter(fp_v, [tidx], evs[nq - 1] * rinv,
                                   mask=tmask)
                return 0

            lax.fori_loop(0, cb, row_body, 0)

            pltpu.sync_copy(fs_v, scores_hbm.at[pl.ds(fbase, g_rows)])
            pltpu.sync_copy(fp_v, post_hbm.at[pl.ds(fbase, g_rows)])
            return 0

        lax.fori_loop(0, n_chunks, chunk_body, 0)

    return entity_kernel


def kernel(context_encoded, entity_ids, knwn_entity_embeddings):
    info = plsc.get_sparse_core_info()
    nc, ns, nl = info.num_cores, info.num_subcores, info.num_lanes
    ids2d = entity_ids.astype(jnp.int32).reshape(B * N // IDX_MINOR, IDX_MINOR)
    k = _entity_kernel(nc, ns, nl)
    scores_flat, post_flat = k(ids2d, context_encoded, knwn_entity_embeddings)
    return scores_flat.reshape(B, N), post_flat.reshape(B, N)
